# Initial kernel scaffold; baseline (speedup 1.0000x reference)
import jax
import jax.numpy as jnp
from jax.experimental import pallas as pl


def kernel(x, edge_index, params):
    raise NotImplementedError("write your pallas kernel here")


# speedup vs baseline: 12.2590x; 12.2590x over previous
import functools

import numpy as np
import jax
import jax.numpy as jnp
from jax import lax
from jax.experimental import pallas as pl
from jax.experimental.pallas import tpu as pltpu
from jax.experimental.pallas import tpu_sc as plsc

_B, _T, _N, _F = 16, 24, 300, 32
_HID, _FF, _HEADS, _OC, _OF = 128, 256, 4, 64, 14
_E = 3000
_NP = 304
_EP = 3008
_CSZ = _NP * _NP + 16
_NEG = -1e30
_SCALE = 1.0 / float(np.sqrt(_HID))


def _ln(x, g, b):
    m = jnp.mean(x, -1, keepdims=True)
    d = x - m
    v = jnp.mean(d * d, -1, keepdims=True)
    return d * lax.rsqrt(v + 1e-5) * g + b


def _nt(a, b):
    return lax.dot_general(a, b, (((1,), (1,)), ((), ())))


def _softmax(s):
    m = jnp.max(s, -1, keepdims=True)
    e = jnp.exp(s - m)
    return e / jnp.sum(e, -1, keepdims=True)


def _count_body(edges_hbm, zeros_hbm, out_hbm, s_v, f_v, ones_v, c_sh):
    cid = lax.axis_index("c")
    sid = lax.axis_index("s")

    @pl.when(jnp.logical_and(cid == 0, sid == 0))
    def _():
        pltpu.sync_copy(zeros_hbm, c_sh)
        pltpu.sync_copy(edges_hbm.at[0], s_v)
        pltpu.sync_copy(edges_hbm.at[1], f_v)

        def body(i, carry):
            sl = pl.ds(i * 16, 16)
            f_v[sl] = f_v[sl] * _NP + s_v[sl]
            ones_v[sl] = jnp.ones((16,), jnp.float32)
            return carry

        lax.fori_loop(0, _EP // 16, body, 0)
        pltpu.sync_copy(ones_v, c_sh.at[f_v], add=True)
        pltpu.sync_copy(c_sh, out_hbm)


@functools.cache
def _edge_counts_kernel():
    return pl.kernel(
        _count_body,
        mesh=plsc.VectorSubcoreMesh(core_axis_name="c", subcore_axis_name="s"),
        out_type=jax.ShapeDtypeStruct((_CSZ,), jnp.float32),
        scratch_types=[
            pltpu.VMEM((_EP,), jnp.int32),
            pltpu.VMEM((_EP,), jnp.int32),
            pltpu.VMEM((_EP,), jnp.float32),
            pltpu.VMEM_SHARED((_CSZ,), jnp.float32),
        ],
    )


def _enc_body(x_ref, g1, be1, wqkv, bqkv, wo, bo, wsk, bsk, g2, be2,
              wkv, bkv, wq2, bq2, wo2, bo2, g3, be3, wf1, bf1, wf2, bf2,
              out_ref, x1_s, ks_s, vs_s, s2_s):
    tril = (lax.broadcasted_iota(jnp.int32, (_NP, _NP), 0)
            >= lax.broadcasted_iota(jnp.int32, (_NP, _NP), 1))
    Wqkv = wqkv[...]; Bqkv = bqkv[...]
    Wo = wo[...]; Bo = bo[...]
    Wsk = wsk[...]; Bsk = bsk[...]
    G1 = g1[...]; Be1 = be1[...]
    G2 = g2[...]; Be2 = be2[...]
    Wkv = wkv[...]; Bkv = bkv[...]

    def step(t, carry):
        xt = x_ref[0, t]
        h = _ln(xt, G1, Be1)
        qkv = h @ Wqkv + Bqkv
        q = qkv[:, :_HID]
        k = qkv[:, _HID:2 * _HID]
        v = qkv[:, 2 * _HID:]
        s = jnp.where(tril, _nt(q, k) * _SCALE, _NEG)
        p = _softmax(s)
        x1 = xt @ Wsk + Bsk + (p @ v) @ Wo + Bo
        x1_s[t] = x1
        h2 = _ln(x1, G2, Be2)
        kv = h2 @ Wkv + Bkv
        ks_s[t] = kv[:, :_HID]
        vs_s[t] = kv[:, _HID:]
        return carry

    lax.fori_loop(0, _T, step, 0)

    x1l = x1_s[_T - 1]
    h2 = _ln(x1l, G2, Be2)
    q2 = (h2 @ wq2[...] + bq2[...]) * _SCALE
    for t in range(_T):
        s2_s[:, t:t + 1] = jnp.sum(q2 * ks_s[t], -1, keepdims=True)
    lane = lax.broadcasted_iota(jnp.int32, (_NP, _HID), 1)
    p2 = _softmax(jnp.where(lane < _T, s2_s[...], _NEG))
    acc = jnp.zeros((_NP, _HID), jnp.float32)
    for t in range(_T):
        acc = acc + p2[:, t:t + 1] * vs_s[t]
    x2 = x1l + acc @ wo2[...] + bo2[...]
    h3 = _ln(x2, g3[...], be3[...])
    f = jnp.maximum(h3 @ wf1[...] + bf1[...], 0.0)
    out_ref[0] = x2 + f @ wf2[...] + bf2[...]


def _gat_body(h_ref, c_ref, wg1, ar_s1, ar_d1, bg1, wg2, ar_s2, ar_d2, bg2,
              wf, bf, out_ref):
    ii = lax.broadcasted_iota(jnp.int32, (_NP, _NP), 0)
    jj = lax.broadcasted_iota(jnp.int32, (_NP, _NP), 1)
    C = c_ref[...] + jnp.where(ii == jj, 1.0, 0.0)
    edge = C > 0
    h = h_ref[0]

    def gat(hg, asr, adc):
        A = adc + asr
        A = jnp.where(A > 0, A, 0.2 * A)
        amax = jnp.max(jnp.where(edge, A, _NEG), -1, keepdims=True)
        e = jnp.where(edge, C * jnp.exp(A - amax), 0.0)
        P = e / (jnp.sum(e, -1, keepdims=True) + 1e-16)
        return P @ hg

    hg = h @ wg1[...]
    adc = _nt(hg, ar_d1[...])
    asr = _nt(ar_s1[...], hg)
    ch = _HID // _HEADS
    outs = [gat(hg[:, k * ch:(k + 1) * ch],
                asr[k:k + 1, :], adc[:, k:k + 1]) for k in range(_HEADS)]
    g = jnp.concatenate(outs, axis=1) + bg1[...]
    h1 = jnp.where(g > 0, g, jnp.exp(g) - 1.0)

    hg2 = h1 @ wg2[...]
    o2 = gat(hg2, _nt(ar_s2[...], hg2), _nt(hg2, ar_d2[...])) + bg2[...]
    out_ref[0] = o2 @ wf[...] + bf[...]


def _full_spec(shape):
    nd = len(shape)
    return pl.BlockSpec(shape, lambda b, _n=nd: (0,) * _n)


def _encoder_call(xp, wlist, interpret=False):
    in_specs = [pl.BlockSpec((1, _T, _NP, _F), lambda b: (b, 0, 0, 0))]
    in_specs += [_full_spec(w.shape) for w in wlist]
    return pl.pallas_call(
        _enc_body,
        grid=(_B,),
        in_specs=in_specs,
        out_specs=pl.BlockSpec((1, _NP, _HID), lambda b: (b, 0, 0)),
        out_shape=jax.ShapeDtypeStruct((_B, _NP, _HID), jnp.float32),
        scratch_shapes=[
            pltpu.VMEM((_T, _NP, _HID), jnp.float32),
            pltpu.VMEM((_T, _NP, _HID), jnp.float32),
            pltpu.VMEM((_T, _NP, _HID), jnp.float32),
            pltpu.VMEM((_NP, _HID), jnp.float32),
        ],
        interpret=interpret,
    )(xp, *wlist)


def _gat_call(hl, C, wlist, interpret=False):
    in_specs = [pl.BlockSpec((1, _NP, _HID), lambda b: (b, 0, 0)),
                _full_spec((_NP, _NP))]
    in_specs += [_full_spec(w.shape) for w in wlist]
    return pl.pallas_call(
        _gat_body,
        grid=(_B,),
        in_specs=in_specs,
        out_specs=pl.BlockSpec((1, _NP, _OF), lambda b: (b, 0, 0)),
        out_shape=jax.ShapeDtypeStruct((_B, _NP, _OF), jnp.float32),
        interpret=interpret,
    )(hl, C, *wlist)


def _head_rows(a):
    heads, ch = a.shape
    return (jnp.eye(heads, dtype=a.dtype)[:, :, None] * a[None, :, :]) \
        .reshape(heads, heads * ch)


def _prep_weights(p):
    r1 = lambda v: v.reshape(1, -1)
    enc = [r1(p['g1']), r1(p['be1']),
           jnp.concatenate([p['Wq_t'], p['Wk_t'], p['Wv_t']], axis=1),
           jnp.concatenate([p['bq_t'], p['bk_t'], p['bv_t']]).reshape(1, -1),
           p['Wo_t'], r1(p['bo_t']), p['W_skip'], r1(p['b_skip']),
           r1(p['g2']), r1(p['be2']),
           jnp.concatenate([p['Wk_s'], p['Wv_s']], axis=1),
           jnp.concatenate([p['bk_s'], p['bv_s']]).reshape(1, -1),
           p['Wq_s'], r1(p['bq_s']), p['Wo_s'], r1(p['bo_s']),
           r1(p['g3']), r1(p['be3']),
           p['W_ff1'], r1(p['b_ff1']), p['W_ff2'], r1(p['b_ff2'])]
    gat = [p['W_g1'], _head_rows(p['a_s1']), _head_rows(p['a_d1']),
           r1(p['b_g1']), p['W_g2'], p['a_s2'], p['a_d2'], r1(p['b_g2']),
           p['W_f'], r1(p['b_f'])]
    return enc, gat


def kernel(x, edge_index, params):
    xp = jnp.pad(x, ((0, 0), (0, 0), (0, _NP - _N), (0, 0)))
    ei = edge_index.astype(jnp.int32)
    pad = jnp.concatenate(
        [jnp.zeros((1, _EP - _E), jnp.int32),
         jnp.full((1, _EP - _E), _NP, jnp.int32)], axis=0)
    ep = jnp.concatenate([ei, pad], axis=1)

    C = _edge_counts_kernel()(ep, jnp.zeros((_CSZ,), jnp.float32))
    C = C[:_NP * _NP].reshape(_NP, _NP)

    enc_w, gat_w = _prep_weights(params)
    hl = _encoder_call(xp, enc_w)
    out = _gat_call(hl, C, gat_w)
    return jnp.transpose(out[:, :_N, :], (0, 2, 1))


# speedup vs baseline: 15.8388x; 1.2920x over previous
import functools

import numpy as np
import jax
import jax.numpy as jnp
from jax import lax
from jax.experimental import pallas as pl
from jax.experimental.pallas import tpu as pltpu
from jax.experimental.pallas import tpu_sc as plsc

_B, _T, _N, _F = 16, 24, 300, 32
_HID, _FF, _HEADS, _OC, _OF = 128, 256, 4, 64, 14
_E = 3000
_NP = 304
_EP = 3008
_CSZ = _NP * _NP + 16
_NEG = -1e30
_SCALE = 1.0 / float(np.sqrt(_HID))


def _ln(x, g, b):
    m = jnp.mean(x, -1, keepdims=True)
    d = x - m
    v = jnp.mean(d * d, -1, keepdims=True)
    return d * lax.rsqrt(v + 1e-5) * g + b


def _nt(a, b):
    return lax.dot_general(a, b, (((1,), (1,)), ((), ())))


def _softmax(s):
    m = jnp.max(s, -1, keepdims=True)
    e = jnp.exp(s - m)
    return e / jnp.sum(e, -1, keepdims=True)


def _count_body(edges_hbm, zeros_hbm, out_hbm, s_v, f_v, ones_v, c_sh):
    cid = lax.axis_index("c")
    sid = lax.axis_index("s")

    @pl.when(jnp.logical_and(cid == 0, sid == 0))
    def _():
        pltpu.sync_copy(zeros_hbm, c_sh)
        pltpu.sync_copy(edges_hbm.at[0], s_v)
        pltpu.sync_copy(edges_hbm.at[1], f_v)

        def body(i, carry):
            sl = pl.ds(i * 16, 16)
            f_v[sl] = f_v[sl] * _NP + s_v[sl]
            ones_v[sl] = jnp.ones((16,), jnp.float32)
            return carry

        lax.fori_loop(0, _EP // 16, body, 0)
        pltpu.sync_copy(ones_v, c_sh.at[f_v], add=True)
        pltpu.sync_copy(c_sh, out_hbm)


@functools.cache
def _edge_counts_kernel():
    return pl.kernel(
        _count_body,
        mesh=plsc.VectorSubcoreMesh(core_axis_name="c", subcore_axis_name="s"),
        out_type=jax.ShapeDtypeStruct((_CSZ,), jnp.float32),
        scratch_types=[
            pltpu.VMEM((_EP,), jnp.int32),
            pltpu.VMEM((_EP,), jnp.int32),
            pltpu.VMEM((_EP,), jnp.float32),
            pltpu.VMEM_SHARED((_CSZ,), jnp.float32),
        ],
    )


def _bf(x):
    return x.astype(jnp.bfloat16)


def _mm(a, b):
    return jnp.dot(_bf(a), _bf(b), preferred_element_type=jnp.float32)


def _ntf(a, b):
    return lax.dot_general(_bf(a), _bf(b), (((1,), (1,)), ((), ())),
                           preferred_element_type=jnp.float32)


def _enc_body(x_ref, g1, be1, wqkv, bqkv, wo, bo, wsk, bsk, g2, be2,
              wkv, bkv, wq2, bq2, wo2, bo2, g3, be3, wf1, bf1, wf2, bf2,
              out_ref, qkv_s, o_s, kv_s, s2_s):
    tril = (lax.broadcasted_iota(jnp.int32, (_NP, _NP), 0)
            >= lax.broadcasted_iota(jnp.int32, (_NP, _NP), 1))
    x2d = x_ref[0].reshape(_T * _NP, _F)
    h = _ln(x2d, g1[...], be1[...])
    qkv_s[...] = (_mm(h, wqkv[...]) + bqkv[...]) \
        .astype(jnp.bfloat16).reshape(_T, _NP, 3 * _HID)

    def step(t, carry):
        q = qkv_s[t, :, :_HID]
        k = qkv_s[t, :, _HID:2 * _HID]
        v = qkv_s[t, :, 2 * _HID:]
        s = jnp.where(tril, _ntf(q, k) * _SCALE, _NEG)
        p = _softmax(s)
        o_s[t] = jnp.dot(_bf(p), v, preferred_element_type=jnp.float32) \
            .astype(jnp.bfloat16)
        return carry

    lax.fori_loop(0, _T, step, 0)

    o2d = o_s[...].reshape(_T * _NP, _HID)
    x1 = _mm(x2d, wsk[...]) + bsk[...] + \
        jnp.dot(o2d, _bf(wo[...]), preferred_element_type=jnp.float32) + bo[...]
    h2 = _ln(x1, g2[...], be2[...])
    kv_s[...] = _mm(h2, wkv[...]) + bkv[...]

    x1l = x1[(_T - 1) * _NP:, :]
    h2l = h2[(_T - 1) * _NP:, :]
    q2 = (_mm(h2l, wq2[...]) + bq2[...]) * _SCALE
    for t in range(_T):
        s2_s[:, t:t + 1] = jnp.sum(
            q2 * kv_s[t * _NP:(t + 1) * _NP, :_HID], -1, keepdims=True)
    lane = lax.broadcasted_iota(jnp.int32, (_NP, _HID), 1)
    p2 = _softmax(jnp.where(lane < _T, s2_s[...], _NEG))
    acc = jnp.zeros((_NP, _HID), jnp.float32)
    for t in range(_T):
        acc = acc + p2[:, t:t + 1] * kv_s[t * _NP:(t + 1) * _NP, _HID:]
    x2 = x1l + _mm(acc, wo2[...]) + bo2[...]
    h3 = _ln(x2, g3[...], be3[...])
    f = jnp.maximum(_mm(h3, wf1[...]) + bf1[...], 0.0)
    out_ref[0] = x2 + _mm(f, wf2[...]) + bf2[...]


def _gat_body(h_ref, c_ref, wg1, ar_s1, ar_d1, bg1, wg2, ar_s2, ar_d2, bg2,
              wf, bf, out_ref):
    ii = lax.broadcasted_iota(jnp.int32, (_NP, _NP), 0)
    jj = lax.broadcasted_iota(jnp.int32, (_NP, _NP), 1)
    C = c_ref[...] + jnp.where(ii == jj, 1.0, 0.0)
    edge = C > 0
    h = h_ref[0]

    def gat(hg, asr, adc):
        A = adc + asr
        A = jnp.where(A > 0, A, 0.2 * A)
        amax = jnp.max(jnp.where(edge, A, _NEG), -1, keepdims=True)
        e = jnp.where(edge, C * jnp.exp(A - amax), 0.0)
        P = e / (jnp.sum(e, -1, keepdims=True) + 1e-16)
        return P @ hg

    hg = h @ wg1[...]
    adc = _nt(hg, ar_d1[...])
    asr = _nt(ar_s1[...], hg)
    ch = _HID // _HEADS
    outs = [gat(hg[:, k * ch:(k + 1) * ch],
                asr[k:k + 1, :], adc[:, k:k + 1]) for k in range(_HEADS)]
    g = jnp.concatenate(outs, axis=1) + bg1[...]
    h1 = jnp.where(g > 0, g, jnp.exp(g) - 1.0)

    hg2 = h1 @ wg2[...]
    o2 = gat(hg2, _nt(ar_s2[...], hg2), _nt(hg2, ar_d2[...])) + bg2[...]
    out_ref[0] = o2 @ wf[...] + bf[...]


def _full_spec(shape):
    nd = len(shape)
    return pl.BlockSpec(shape, lambda b, _n=nd: (0,) * _n)


def _encoder_call(xp, wlist, interpret=False):
    in_specs = [pl.BlockSpec((1, _T, _NP, _F), lambda b: (b, 0, 0, 0))]
    in_specs += [_full_spec(w.shape) for w in wlist]
    return pl.pallas_call(
        _enc_body,
        grid=(_B,),
        in_specs=in_specs,
        out_specs=pl.BlockSpec((1, _NP, _HID), lambda b: (b, 0, 0)),
        out_shape=jax.ShapeDtypeStruct((_B, _NP, _HID), jnp.float32),
        scratch_shapes=[
            pltpu.VMEM((_T, _NP, 3 * _HID), jnp.bfloat16),
            pltpu.VMEM((_T, _NP, _HID), jnp.bfloat16),
            pltpu.VMEM((_T * _NP, 2 * _HID), jnp.float32),
            pltpu.VMEM((_NP, _HID), jnp.float32),
        ],
        interpret=interpret,
    )(xp, *wlist)


def _gat_call(hl, C, wlist, interpret=False):
    in_specs = [pl.BlockSpec((1, _NP, _HID), lambda b: (b, 0, 0)),
                _full_spec((_NP, _NP))]
    in_specs += [_full_spec(w.shape) for w in wlist]
    return pl.pallas_call(
        _gat_body,
        grid=(_B,),
        in_specs=in_specs,
        out_specs=pl.BlockSpec((1, _NP, _OF), lambda b: (b, 0, 0)),
        out_shape=jax.ShapeDtypeStruct((_B, _NP, _OF), jnp.float32),
        interpret=interpret,
    )(hl, C, *wlist)


def _head_rows(a):
    heads, ch = a.shape
    return (jnp.eye(heads, dtype=a.dtype)[:, :, None] * a[None, :, :]) \
        .reshape(heads, heads * ch)


def _prep_weights(p):
    r1 = lambda v: v.reshape(1, -1)
    enc = [r1(p['g1']), r1(p['be1']),
           jnp.concatenate([p['Wq_t'], p['Wk_t'], p['Wv_t']], axis=1),
           jnp.concatenate([p['bq_t'], p['bk_t'], p['bv_t']]).reshape(1, -1),
           p['Wo_t'], r1(p['bo_t']), p['W_skip'], r1(p['b_skip']),
           r1(p['g2']), r1(p['be2']),
           jnp.concatenate([p['Wk_s'], p['Wv_s']], axis=1),
           jnp.concatenate([p['bk_s'], p['bv_s']]).reshape(1, -1),
           p['Wq_s'], r1(p['bq_s']), p['Wo_s'], r1(p['bo_s']),
           r1(p['g3']), r1(p['be3']),
           p['W_ff1'], r1(p['b_ff1']), p['W_ff2'], r1(p['b_ff2'])]
    gat = [p['W_g1'], _head_rows(p['a_s1']), _head_rows(p['a_d1']),
           r1(p['b_g1']), p['W_g2'], p['a_s2'], p['a_d2'], r1(p['b_g2']),
           p['W_f'], r1(p['b_f'])]
    return enc, gat


def kernel(x, edge_index, params):
    xp = jnp.pad(x, ((0, 0), (0, 0), (0, _NP - _N), (0, 0)))
    ei = edge_index.astype(jnp.int32)
    pad = jnp.concatenate(
        [jnp.zeros((1, _EP - _E), jnp.int32),
         jnp.full((1, _EP - _E), _NP, jnp.int32)], axis=0)
    ep = jnp.concatenate([ei, pad], axis=1)

    C = _edge_counts_kernel()(ep, jnp.zeros((_CSZ,), jnp.float32))
    C = C[:_NP * _NP].reshape(_NP, _NP)

    enc_w, gat_w = _prep_weights(params)
    hl = _encoder_call(xp, enc_w)
    out = _gat_call(hl, C, gat_w)
    return jnp.transpose(out[:, :_N, :], (0, 2, 1))


# speedup vs baseline: 18.8292x; 1.1888x over previous
import functools

import numpy as np
import jax
import jax.numpy as jnp
from jax import lax
from jax.experimental import pallas as pl
from jax.experimental.pallas import tpu as pltpu
from jax.experimental.pallas import tpu_sc as plsc

_B, _T, _N, _F = 16, 24, 300, 32
_HID, _FF, _HEADS, _OC, _OF = 128, 256, 4, 64, 14
_E = 3000
_NP = 304
_EP = 3008
_CSZ = _NP * _NP + 16
_NEG = -1e30
_SCALE = 1.0 / float(np.sqrt(_HID))


def _ln(x, g, b):
    ones = jnp.full((x.shape[-1], 1), 1.0 / x.shape[-1], jnp.float32)
    m = jnp.dot(x, ones, preferred_element_type=jnp.float32)
    ex2 = jnp.dot(x * x, ones, preferred_element_type=jnp.float32)
    v = ex2 - m * m
    return (x - m) * lax.rsqrt(v + 1e-5) * g + b


def _nt(a, b):
    return lax.dot_general(a, b, (((1,), (1,)), ((), ())))


def _softmax(s):
    m = jnp.max(s, -1, keepdims=True)
    e = jnp.exp(s - m)
    return e / jnp.sum(e, -1, keepdims=True)


def _count_body(edges_hbm, zeros_hbm, out_hbm, s_v, f_v, ones_v, c_sh):
    cid = lax.axis_index("c")
    sid = lax.axis_index("s")

    @pl.when(jnp.logical_and(cid == 0, sid == 0))
    def _():
        pltpu.sync_copy(zeros_hbm, c_sh)
        pltpu.sync_copy(edges_hbm.at[0], s_v)
        pltpu.sync_copy(edges_hbm.at[1], f_v)

        def body(i, carry):
            sl = pl.ds(i * 16, 16)
            f_v[sl] = f_v[sl] * _NP + s_v[sl]
            ones_v[sl] = jnp.ones((16,), jnp.float32)
            return carry

        lax.fori_loop(0, _EP // 16, body, 0)
        pltpu.sync_copy(ones_v, c_sh.at[f_v], add=True)
        pltpu.sync_copy(c_sh, out_hbm)


@functools.cache
def _edge_counts_kernel():
    return pl.kernel(
        _count_body,
        mesh=plsc.VectorSubcoreMesh(core_axis_name="c", subcore_axis_name="s"),
        out_type=jax.ShapeDtypeStruct((_CSZ,), jnp.float32),
        scratch_types=[
            pltpu.VMEM((_EP,), jnp.int32),
            pltpu.VMEM((_EP,), jnp.int32),
            pltpu.VMEM((_EP,), jnp.float32),
            pltpu.VMEM_SHARED((_CSZ,), jnp.float32),
        ],
    )


def _bf(x):
    return x.astype(jnp.bfloat16)


def _mm(a, b):
    return jnp.dot(_bf(a), _bf(b), preferred_element_type=jnp.float32)


def _ntf(a, b):
    return lax.dot_general(_bf(a), _bf(b), (((1,), (1,)), ((), ())),
                           preferred_element_type=jnp.float32)


def _enc_body(x_ref, g1, be1, wqkv, bqkv, wo, bo, wsk, bsk, g2, be2,
              wkv, bkv, wq2, bq2, wo2, bo2, g3, be3, wf1, bf1, wf2, bf2,
              out_ref, qkv_s, o_s, kv_s, s2_s):
    tril = (lax.broadcasted_iota(jnp.int32, (_NP, _NP), 0)
            >= lax.broadcasted_iota(jnp.int32, (_NP, _NP), 1))
    x2d = x_ref[0].reshape(_T * _NP, _F)
    h = _ln(x2d, g1[...], be1[...])
    qkv_s[...] = (_mm(h, wqkv[...]) + bqkv[...]) \
        .astype(jnp.bfloat16).reshape(_T, _NP, 3 * _HID)

    def step(t, carry):
        q = qkv_s[t, :, :_HID]
        k = qkv_s[t, :, _HID:2 * _HID]
        v = qkv_s[t, :, 2 * _HID:]
        e = jnp.exp(jnp.where(tril, _ntf(q, k) * _SCALE, _NEG))
        o = jnp.dot(_bf(e), v, preferred_element_type=jnp.float32)
        o_s[t] = (o / jnp.sum(e, -1, keepdims=True)).astype(jnp.bfloat16)
        return carry

    lax.fori_loop(0, _T, step, 0)

    o2d = o_s[...].reshape(_T * _NP, _HID)
    x1 = _mm(x2d, wsk[...]) + bsk[...] + \
        jnp.dot(o2d, _bf(wo[...]), preferred_element_type=jnp.float32) + bo[...]
    h2 = _ln(x1, g2[...], be2[...])
    kv_s[...] = _mm(h2, wkv[...]) + bkv[...]

    x1l = x1[(_T - 1) * _NP:, :]
    h2l = h2[(_T - 1) * _NP:, :]
    q2 = (_mm(h2l, wq2[...]) + bq2[...]) * _SCALE
    ones_h = jnp.full((_HID, 1), 1.0, jnp.float32)
    for t in range(_T):
        s2_s[:, t:t + 1] = jnp.dot(
            q2 * kv_s[t * _NP:(t + 1) * _NP, :_HID], ones_h,
            preferred_element_type=jnp.float32)
    lane = lax.broadcasted_iota(jnp.int32, (_NP, _HID), 1)
    p2 = _softmax(jnp.where(lane < _T, s2_s[...], _NEG))
    acc = jnp.zeros((_NP, _HID), jnp.float32)
    for t in range(_T):
        acc = acc + p2[:, t:t + 1] * kv_s[t * _NP:(t + 1) * _NP, _HID:]
    x2 = x1l + _mm(acc, wo2[...]) + bo2[...]
    h3 = _ln(x2, g3[...], be3[...])
    f = jnp.maximum(_mm(h3, wf1[...]) + bf1[...], 0.0)
    out_ref[0] = x2 + _mm(f, wf2[...]) + bf2[...]


def _gat_body(h_ref, c_ref, wg1, ar_s1, ar_d1, bg1, wg2, ar_s2, ar_d2, bg2,
              wf, bf, out_ref):
    ii = lax.broadcasted_iota(jnp.int32, (_NP, _NP), 0)
    jj = lax.broadcasted_iota(jnp.int32, (_NP, _NP), 1)
    C = c_ref[...] + jnp.where(ii == jj, 1.0, 0.0)
    edge = C > 0
    h = h_ref[0]

    def gat(hg, asr, adc):
        A = adc + asr
        A = jnp.where(A > 0, A, 0.2 * A)
        e = jnp.where(edge, C * jnp.exp(A), 0.0)
        o = jnp.dot(_bf(e), _bf(hg), preferred_element_type=jnp.float32)
        return o / (jnp.sum(e, -1, keepdims=True) + 1e-16)

    hg = h @ wg1[...]
    adc = _nt(hg, ar_d1[...])
    asr = _nt(ar_s1[...], hg)
    ch = _HID // _HEADS
    outs = [gat(hg[:, k * ch:(k + 1) * ch],
                asr[k:k + 1, :], adc[:, k:k + 1]) for k in range(_HEADS)]
    g = jnp.concatenate(outs, axis=1) + bg1[...]
    h1 = jnp.where(g > 0, g, jnp.exp(g) - 1.0)

    hg2 = h1 @ wg2[...]
    o2 = gat(hg2, _nt(ar_s2[...], hg2), _nt(hg2, ar_d2[...])) + bg2[...]
    out_ref[0] = o2 @ wf[...] + bf[...]


def _full_spec(shape):
    nd = len(shape)
    return pl.BlockSpec(shape, lambda b, _n=nd: (0,) * _n)


def _encoder_call(xp, wlist, interpret=False):
    in_specs = [pl.BlockSpec((1, _T, _NP, _F), lambda b: (b, 0, 0, 0))]
    in_specs += [_full_spec(w.shape) for w in wlist]
    return pl.pallas_call(
        _enc_body,
        grid=(_B,),
        in_specs=in_specs,
        out_specs=pl.BlockSpec((1, _NP, _HID), lambda b: (b, 0, 0)),
        out_shape=jax.ShapeDtypeStruct((_B, _NP, _HID), jnp.float32),
        scratch_shapes=[
            pltpu.VMEM((_T, _NP, 3 * _HID), jnp.bfloat16),
            pltpu.VMEM((_T, _NP, _HID), jnp.bfloat16),
            pltpu.VMEM((_T * _NP, 2 * _HID), jnp.float32),
            pltpu.VMEM((_NP, _HID), jnp.float32),
        ],
        interpret=interpret,
    )(xp, *wlist)


def _gat_call(hl, C, wlist, interpret=False):
    in_specs = [pl.BlockSpec((1, _NP, _HID), lambda b: (b, 0, 0)),
                _full_spec((_NP, _NP))]
    in_specs += [_full_spec(w.shape) for w in wlist]
    return pl.pallas_call(
        _gat_body,
        grid=(_B,),
        in_specs=in_specs,
        out_specs=pl.BlockSpec((1, _NP, _OF), lambda b: (b, 0, 0)),
        out_shape=jax.ShapeDtypeStruct((_B, _NP, _OF), jnp.float32),
        interpret=interpret,
    )(hl, C, *wlist)


def _head_rows(a):
    heads, ch = a.shape
    return (jnp.eye(heads, dtype=a.dtype)[:, :, None] * a[None, :, :]) \
        .reshape(heads, heads * ch)


def _prep_weights(p):
    r1 = lambda v: v.reshape(1, -1)
    enc = [r1(p['g1']), r1(p['be1']),
           jnp.concatenate([p['Wq_t'], p['Wk_t'], p['Wv_t']], axis=1),
           jnp.concatenate([p['bq_t'], p['bk_t'], p['bv_t']]).reshape(1, -1),
           p['Wo_t'], r1(p['bo_t']), p['W_skip'], r1(p['b_skip']),
           r1(p['g2']), r1(p['be2']),
           jnp.concatenate([p['Wk_s'], p['Wv_s']], axis=1),
           jnp.concatenate([p['bk_s'], p['bv_s']]).reshape(1, -1),
           p['Wq_s'], r1(p['bq_s']), p['Wo_s'], r1(p['bo_s']),
           r1(p['g3']), r1(p['be3']),
           p['W_ff1'], r1(p['b_ff1']), p['W_ff2'], r1(p['b_ff2'])]
    gat = [p['W_g1'], _head_rows(p['a_s1']), _head_rows(p['a_d1']),
           r1(p['b_g1']), p['W_g2'], p['a_s2'], p['a_d2'], r1(p['b_g2']),
           p['W_f'], r1(p['b_f'])]
    return enc, gat


def kernel(x, edge_index, params):
    xp = jnp.pad(x, ((0, 0), (0, 0), (0, _NP - _N), (0, 0)))
    ei = edge_index.astype(jnp.int32)
    pad = jnp.concatenate(
        [jnp.zeros((1, _EP - _E), jnp.int32),
         jnp.full((1, _EP - _E), _NP, jnp.int32)], axis=0)
    ep = jnp.concatenate([ei, pad], axis=1)

    C = _edge_counts_kernel()(ep, jnp.zeros((_CSZ,), jnp.float32))
    C = C[:_NP * _NP].reshape(_NP, _NP)

    enc_w, gat_w = _prep_weights(params)
    hl = _encoder_call(xp, enc_w)
    out = _gat_call(hl, C, gat_w)
    return jnp.transpose(out[:, :_N, :], (0, 2, 1))
